# baseline (speedup 1.0000x reference)
import functools

import jax
import jax.numpy as jnp
from jax import lax
from jax.experimental import pallas as pl
from jax.experimental.pallas import tpu as pltpu
from jax.experimental.pallas import tpu_sc as plsc

HIST = 200
HIST_PAD = 256
D = 64
DEPTH = 10

_mesh = plsc.VectorSubcoreMesh(
    core_axis_name="c", subcore_axis_name="s", num_cores=2, num_subcores=16
)


def _sc_body(
    ids_hbm, hist_hbm, user_tt, item_tt,
    w1_hbm, b1_hbm, w2_hbm, b2_hbm, w3_hbm,
    out_hbm,
    idx_v, b0, b1x, b2x, b3x, b4, b5, b6, b7, b8, b9, bufU, bufI, parts_v, allp_v,
    w1_v, b1_v, w2_v, b2_v, w3_v,
    cat_v, h1_v, out_v,
    spart,
    sem_g, sem_u, sem_w,
):
    c = lax.axis_index("c")
    s = lax.axis_index("s")
    iota = lax.iota(jnp.int32, 16)
    bufs = (b0, b1x, b2x, b3x, b4, b5, b6, b7, b8, b9)

    def fetch(table, rid, buf, sem):
        base = pl.multiple_of(rid & -128, 128)
        return pltpu.async_copy(table.at[:, pl.ds(base, 128)], buf, sem)

    wcopies = (
        (w1_hbm, w1_v), (b1_hbm, b1_v), (w2_hbm, w2_v), (b2_hbm, b2_v),
        (w3_hbm, w3_v),
    )

    @pl.when(jnp.logical_and(c == 0, s == 0))
    def _prefetch_phase():
        pltpu.sync_copy(ids_hbm, idx_v)
        ivec = idx_v[...]
        fetch(user_tt, ivec[0], bufU, sem_u)
        fetch(item_tt, ivec[1], bufI, sem_u)
        for src, dst in wcopies:
            pltpu.async_copy(src, dst, sem_w)

    @pl.when(c == 0)
    def _gather_phase():
        pltpu.sync_copy(hist_hbm.at[pl.ds(s * 16, 16)], idx_v)
        ivec = idx_v[...]
        acc = [jnp.zeros((16,), jnp.float32) for _ in range(4)]
        cps = [None] * 16
        for j in range(DEPTH):
            cps[j] = fetch(item_tt, ivec[j], bufs[j], sem_g)
        for j in range(16):
            cps[j].wait()
            lane = jnp.full((16,), ivec[j] & 127, jnp.int32)
            mf = (s * 16 + j < HIST).astype(jnp.float32)
            for i in range(4):
                col = plsc.load_gather(bufs[j % DEPTH], [iota + (i * 16), lane])
                acc[i] = acc[i] + mf * col
            if j + DEPTH < 16:
                cps[j + DEPTH] = fetch(
                    item_tt, ivec[j + DEPTH], bufs[(j + DEPTH) % DEPTH], sem_g
                )
        for i in range(4):
            parts_v[pl.ds(i * 16, 16)] = acc[i]
        pltpu.sync_copy(parts_v, spart.at[s])

    plsc.subcore_barrier()

    @pl.when(jnp.logical_and(c == 0, s == 0))
    def _finish_phase():
        pltpu.sync_copy(spart, allp_v)
        acc = [jnp.zeros((16,), jnp.float32) for _ in range(4)]
        for j in range(16):
            for i in range(4):
                acc[i] = acc[i] + allp_v[j, pl.ds(i * 16, 16)]
        pltpu.make_async_copy(user_tt.at[:, pl.ds(0, 128)], bufU, sem_u).wait()
        pltpu.make_async_copy(item_tt.at[:, pl.ds(0, 128)], bufI, sem_u).wait()
        for src, dst in wcopies:
            pltpu.make_async_copy(src, dst, sem_w).wait()
        pltpu.sync_copy(ids_hbm, idx_v)
        ivec = idx_v[...]
        inv = jnp.float32(1.0 / HIST)
        ulane = jnp.full((16,), ivec[0] & 127, jnp.int32)
        ilane = jnp.full((16,), ivec[1] & 127, jnp.int32)
        for i in range(4):
            ridx = iota + (i * 16)
            cat_v[pl.ds(i * 16, 16)] = plsc.load_gather(bufU, [ridx, ulane])
            cat_v[pl.ds(64 + i * 16, 16)] = plsc.load_gather(bufI, [ridx, ilane])
            cat_v[pl.ds(128 + i * 16, 16)] = acc[i] * inv

        def l1(t, acc):
            cvec = cat_v[pl.ds(t * 16, 16)]
            for j in range(16):
                sval = cvec[j]
                kvec = jnp.full((16,), t * 16 + j, jnp.int32)
                acc = tuple(
                    acc[i] + sval * plsc.load_gather(w1_v, [iota + (i * 16), kvec])
                    for i in range(4)
                )
            return acc

        acc1 = tuple(b1_v[pl.ds(j * 16, 16)] for j in range(4))
        acc1 = lax.fori_loop(0, 12, l1, acc1)
        for j in range(4):
            h1_v[pl.ds(j * 16, 16)] = jnp.maximum(acc1[j], 0.0)

        def l2(t, acc):
            hvec = h1_v[pl.ds(t * 16, 16)]
            for j in range(16):
                sval = hvec[j]
                kvec = jnp.full((16,), t * 16 + j, jnp.int32)
                acc = tuple(
                    acc[i] + sval * plsc.load_gather(w2_v, [iota + (i * 16), kvec])
                    for i in range(2)
                )
            return acc

        acc2 = tuple(b2_v[pl.ds(j * 16, 16)] for j in range(2))
        acc2 = lax.fori_loop(0, 4, l2, acc2)
        h2a = jnp.maximum(acc2[0], 0.0)
        h2b = jnp.maximum(acc2[1], 0.0)

        p = h2a * w3_v[pl.ds(0, 16)] + h2b * w3_v[pl.ds(16, 16)]
        z = w3_v[pl.ds(32, 16)][0]
        for j in range(16):
            z = z + p[j]
        zv = jnp.full((16,), z, jnp.float32)
        out_v[...] = 1.0 / (1.0 + jnp.exp(-zv))
        pltpu.sync_copy(out_v, out_hbm)


_sc_kernel = functools.partial(
    pl.kernel,
    out_type=jax.ShapeDtypeStruct((16,), jnp.float32),
    mesh=_mesh,
    compiler_params=pltpu.CompilerParams(
        use_tc_tiling_on_sc=True, needs_layout_passes=False
    ),
    scratch_types=[
        pltpu.VMEM((16,), jnp.int32),
        pltpu.VMEM((D, 128), jnp.float32),
        pltpu.VMEM((D, 128), jnp.float32),
        pltpu.VMEM((D, 128), jnp.float32),
        pltpu.VMEM((D, 128), jnp.float32),
        pltpu.VMEM((D, 128), jnp.float32),
        pltpu.VMEM((D, 128), jnp.float32),
        pltpu.VMEM((D, 128), jnp.float32),
        pltpu.VMEM((D, 128), jnp.float32),
        pltpu.VMEM((D, 128), jnp.float32),
        pltpu.VMEM((D, 128), jnp.float32),
        pltpu.VMEM((D, 128), jnp.float32),
        pltpu.VMEM((D, 128), jnp.float32),
        pltpu.VMEM((D,), jnp.float32),
        pltpu.VMEM((16, D), jnp.float32),
        pltpu.VMEM((64, 192), jnp.float32),
        pltpu.VMEM((64,), jnp.float32),
        pltpu.VMEM((32, 64), jnp.float32),
        pltpu.VMEM((32,), jnp.float32),
        pltpu.VMEM((48,), jnp.float32),
        pltpu.VMEM((192,), jnp.float32),
        pltpu.VMEM((64,), jnp.float32),
        pltpu.VMEM((16,), jnp.float32),
        pltpu.VMEM_SHARED((16, D), jnp.float32),
        pltpu.SemaphoreType.DMA,
        pltpu.SemaphoreType.DMA,
        pltpu.SemaphoreType.DMA,
    ],
)(_sc_body)


def kernel(user_id, item_history, item_id, user_table, item_table, W1, b1, W2, b2, W3, b3):
    ids = jnp.zeros((16,), jnp.int32)
    ids = ids.at[0].set(user_id.astype(jnp.int32)[0])
    ids = ids.at[1].set(item_id.astype(jnp.int32)[0])
    hist = jnp.zeros((HIST_PAD,), jnp.int32).at[: HIST].set(item_history.astype(jnp.int32))
    w3b = jnp.concatenate(
        [W3.reshape(32).astype(jnp.float32), b3.reshape(1).astype(jnp.float32),
         jnp.zeros((15,), jnp.float32)]
    )
    out16 = _sc_kernel(
        ids, hist, user_table.T, item_table.T, W1, b1, W2, b2, w3b
    )
    return out16[0].reshape(1, 1, 1)


# speedup vs baseline: 1.0734x; 1.0734x over previous
import functools

import jax
import jax.numpy as jnp
from jax import lax
from jax.experimental import pallas as pl
from jax.experimental.pallas import tpu as pltpu
from jax.experimental.pallas import tpu_sc as plsc

HIST = 200
HIST_PAD = 256
D = 64
NIT = 8
DEPTH = 6

_mesh = plsc.VectorSubcoreMesh(
    core_axis_name="c", subcore_axis_name="s", num_cores=2, num_subcores=16
)


def _sc_body(
    ids_hbm, hist_hbm, user_tt, item_tt,
    w1_hbm, b1_hbm, w2_hbm, b2_hbm, w3_hbm,
    out_hbm, x1_hbm,
    idx_v, b0, b1x, b2x, b3x, b4, b5, bufU, bufI, parts_v, allp_v, c1p_v,
    w1_v, b1_v, w2_v, b2_v, w3_v,
    cat_v, h1_v, out_v,
    spart,
    sem_g, sem_u, sem_w, sem_x,
):
    c = lax.axis_index("c")
    s = lax.axis_index("s")
    wid = s * 2 + c
    iota = lax.iota(jnp.int32, 16)
    bufs = (b0, b1x, b2x, b3x, b4, b5)

    def fetch(table, rid, buf, sem):
        base = pl.multiple_of(rid & -128, 128)
        return pltpu.async_copy(table.at[:, pl.ds(base, 128)], buf, sem)

    wcopies = (
        (w1_hbm, w1_v), (b1_hbm, b1_v), (w2_hbm, w2_v), (b2_hbm, b2_v),
        (w3_hbm, w3_v),
    )

    @pl.when(jnp.logical_and(c == 0, s == 0))
    def _prefetch_phase():
        pltpu.sync_copy(ids_hbm, idx_v)
        ivec = idx_v[...]
        fetch(user_tt, ivec[0], bufU, sem_u)
        fetch(item_tt, ivec[1], bufI, sem_u)
        for src, dst in wcopies:
            pltpu.async_copy(src, dst, sem_w)

    def _gather():
        pltpu.sync_copy(hist_hbm.at[pl.ds(wid * NIT, NIT)], idx_v.at[pl.ds(0, NIT)])
        ivec = idx_v[...]
        acc = [jnp.zeros((16,), jnp.float32) for _ in range(4)]
        cps = [None] * NIT
        for j in range(DEPTH):
            cps[j] = fetch(item_tt, ivec[j], bufs[j], sem_g)
        for j in range(NIT):
            cps[j].wait()
            lane = jnp.full((16,), ivec[j] & 127, jnp.int32)
            mf = (wid * NIT + j < HIST).astype(jnp.float32)
            for i in range(4):
                col = plsc.load_gather(bufs[j % DEPTH], [iota + (i * 16), lane])
                acc[i] = acc[i] + mf * col
            if j + DEPTH < NIT:
                cps[j + DEPTH] = fetch(
                    item_tt, ivec[j + DEPTH], bufs[(j + DEPTH) % DEPTH], sem_g
                )
        for i in range(4):
            parts_v[pl.ds(i * 16, 16)] = acc[i]

    @pl.when(wid < 25)
    def _gather_phase():
        _gather()

    @pl.when(wid >= 25)
    def _zero_phase():
        for i in range(4):
            parts_v[pl.ds(i * 16, 16)] = jnp.zeros((16,), jnp.float32)

    pltpu.sync_copy(parts_v, spart.at[s])
    plsc.subcore_barrier()

    @pl.when(jnp.logical_and(c == 1, s == 0))
    def _core1_reduce():
        pltpu.sync_copy(spart, allp_v)
        acc = [jnp.zeros((16,), jnp.float32) for _ in range(4)]
        for j in range(16):
            for i in range(4):
                acc[i] = acc[i] + allp_v[j, pl.ds(i * 16, 16)]
        for i in range(4):
            parts_v[pl.ds(i * 16, 16)] = acc[i]
        pltpu.sync_copy(parts_v, x1_hbm)
        pltpu.semaphore_signal(sem_x, 1, core_index=0)

    @pl.when(jnp.logical_and(c == 0, s == 0))
    def _finish_phase():
        pltpu.sync_copy(spart, allp_v)
        acc = [jnp.zeros((16,), jnp.float32) for _ in range(4)]
        for j in range(16):
            for i in range(4):
                acc[i] = acc[i] + allp_v[j, pl.ds(i * 16, 16)]
        pltpu.make_async_copy(user_tt.at[:, pl.ds(0, 128)], bufU, sem_u).wait()
        pltpu.make_async_copy(item_tt.at[:, pl.ds(0, 128)], bufI, sem_u).wait()
        for src, dst in wcopies:
            pltpu.make_async_copy(src, dst, sem_w).wait()
        pltpu.sync_copy(ids_hbm, idx_v)
        ivec = idx_v[...]
        pl.semaphore_wait(sem_x, 1)
        pltpu.sync_copy(x1_hbm, c1p_v)
        for i in range(4):
            acc[i] = acc[i] + c1p_v[pl.ds(i * 16, 16)]
        inv = jnp.float32(1.0 / HIST)
        ulane = jnp.full((16,), ivec[0] & 127, jnp.int32)
        ilane = jnp.full((16,), ivec[1] & 127, jnp.int32)
        for i in range(4):
            ridx = iota + (i * 16)
            cat_v[pl.ds(i * 16, 16)] = plsc.load_gather(bufU, [ridx, ulane])
            cat_v[pl.ds(64 + i * 16, 16)] = plsc.load_gather(bufI, [ridx, ilane])
            cat_v[pl.ds(128 + i * 16, 16)] = acc[i] * inv

        def l1(t, acc):
            cvec = cat_v[pl.ds(t * 16, 16)]
            for j in range(16):
                sval = cvec[j]
                kvec = jnp.full((16,), t * 16 + j, jnp.int32)
                acc = tuple(
                    acc[i] + sval * plsc.load_gather(w1_v, [iota + (i * 16), kvec])
                    for i in range(4)
                )
            return acc

        acc1 = tuple(b1_v[pl.ds(j * 16, 16)] for j in range(4))
        acc1 = lax.fori_loop(0, 12, l1, acc1)
        for j in range(4):
            h1_v[pl.ds(j * 16, 16)] = jnp.maximum(acc1[j], 0.0)

        def l2(t, acc):
            hvec = h1_v[pl.ds(t * 16, 16)]
            for j in range(16):
                sval = hvec[j]
                kvec = jnp.full((16,), t * 16 + j, jnp.int32)
                acc = tuple(
                    acc[i] + sval * plsc.load_gather(w2_v, [iota + (i * 16), kvec])
                    for i in range(2)
                )
            return acc

        acc2 = tuple(b2_v[pl.ds(j * 16, 16)] for j in range(2))
        acc2 = lax.fori_loop(0, 4, l2, acc2)
        h2a = jnp.maximum(acc2[0], 0.0)
        h2b = jnp.maximum(acc2[1], 0.0)

        p = h2a * w3_v[pl.ds(0, 16)] + h2b * w3_v[pl.ds(16, 16)]
        z = w3_v[pl.ds(32, 16)][0]
        for j in range(16):
            z = z + p[j]
        zv = jnp.full((16,), z, jnp.float32)
        out_v[...] = 1.0 / (1.0 + jnp.exp(-zv))
        pltpu.sync_copy(out_v, out_hbm)


_sc_kernel = functools.partial(
    pl.kernel,
    out_type=(
        jax.ShapeDtypeStruct((16,), jnp.float32),
        jax.ShapeDtypeStruct((D,), jnp.float32),
    ),
    mesh=_mesh,
    compiler_params=pltpu.CompilerParams(
        use_tc_tiling_on_sc=True, needs_layout_passes=False
    ),
    scratch_types=[
        pltpu.VMEM((16,), jnp.int32),
        pltpu.VMEM((D, 128), jnp.float32),
        pltpu.VMEM((D, 128), jnp.float32),
        pltpu.VMEM((D, 128), jnp.float32),
        pltpu.VMEM((D, 128), jnp.float32),
        pltpu.VMEM((D, 128), jnp.float32),
        pltpu.VMEM((D, 128), jnp.float32),
        pltpu.VMEM((D, 128), jnp.float32),
        pltpu.VMEM((D, 128), jnp.float32),
        pltpu.VMEM((D,), jnp.float32),
        pltpu.VMEM((16, D), jnp.float32),
        pltpu.VMEM((D,), jnp.float32),
        pltpu.VMEM((64, 192), jnp.float32),
        pltpu.VMEM((64,), jnp.float32),
        pltpu.VMEM((32, 64), jnp.float32),
        pltpu.VMEM((32,), jnp.float32),
        pltpu.VMEM((48,), jnp.float32),
        pltpu.VMEM((192,), jnp.float32),
        pltpu.VMEM((64,), jnp.float32),
        pltpu.VMEM((16,), jnp.float32),
        pltpu.VMEM_SHARED((16, D), jnp.float32),
        pltpu.SemaphoreType.DMA,
        pltpu.SemaphoreType.DMA,
        pltpu.SemaphoreType.DMA,
        pltpu.SemaphoreType.REGULAR,
    ],
)(_sc_body)


def kernel(user_id, item_history, item_id, user_table, item_table, W1, b1, W2, b2, W3, b3):
    ids = jnp.zeros((16,), jnp.int32)
    ids = ids.at[0].set(user_id.astype(jnp.int32)[0])
    ids = ids.at[1].set(item_id.astype(jnp.int32)[0])
    hist = jnp.zeros((HIST_PAD,), jnp.int32).at[: HIST].set(item_history.astype(jnp.int32))
    w3b = jnp.concatenate(
        [W3.reshape(32).astype(jnp.float32), b3.reshape(1).astype(jnp.float32),
         jnp.zeros((15,), jnp.float32)]
    )
    out16, _ = _sc_kernel(
        ids, hist, user_table.T, item_table.T, W1, b1, W2, b2, w3b
    )
    return out16[0].reshape(1, 1, 1)
